# Initial kernel scaffold; baseline (speedup 1.0000x reference)
import jax
import jax.numpy as jnp
from jax.experimental import pallas as pl


def kernel(state, centroids):
    raise NotImplementedError("write your pallas kernel here")


# speedup vs baseline: 1.1029x; 1.1029x over previous
import jax
import jax.numpy as jnp
from jax.experimental import pallas as pl

_BN = 512
_BK = 1024


def _body(x_ref, c_ref, idx_ref, dist_ref):
    k = pl.program_id(1)
    nk = pl.num_programs(1)
    x = x_ref[...]
    c = c_ref[...]
    dot = jax.lax.dot_general(
        x, c, (((1,), (1,)), ((), ())),
        preferred_element_type=jnp.float32)
    x2 = jnp.sum(x * x, axis=1, keepdims=True)
    c2 = jnp.sum(c * c, axis=1)[None, :]
    d2 = (x2 + c2) - 2.0 * dot
    lmin = jnp.min(d2, axis=1, keepdims=True)
    iota = jax.lax.broadcasted_iota(jnp.int32, d2.shape, 1) + k * _BK
    lidx = jnp.min(
        jnp.where(d2 == lmin, iota, jnp.int32(2**31 - 1)),
        axis=1, keepdims=True)

    @pl.when(k == 0)
    def _init():
        dist_ref[...] = lmin
        idx_ref[...] = lidx

    @pl.when(k > 0)
    def _update():
        better = lmin < dist_ref[...]
        dist_ref[...] = jnp.where(better, lmin, dist_ref[...])
        idx_ref[...] = jnp.where(better, lidx, idx_ref[...])

    @pl.when(k == nk - 1)
    def _finalize():
        dist_ref[...] = jnp.sqrt(jnp.maximum(dist_ref[...], 1e-12))


def kernel(state, centroids):
    if state.ndim == 1:
        state = state[None, :]
    n, d = state.shape
    kk, _ = centroids.shape
    grid = (n // _BN, kk // _BK)
    idx2, dist2 = pl.pallas_call(
        _body,
        grid=grid,
        in_specs=[
            pl.BlockSpec((_BN, d), lambda i, j: (i, 0)),
            pl.BlockSpec((_BK, d), lambda i, j: (j, 0)),
        ],
        out_specs=[
            pl.BlockSpec((_BN, 1), lambda i, j: (i, 0)),
            pl.BlockSpec((_BN, 1), lambda i, j: (i, 0)),
        ],
        out_shape=[
            jax.ShapeDtypeStruct((n, 1), jnp.int32),
            jax.ShapeDtypeStruct((n, 1), jnp.float32),
        ],
    )(state, centroids)
    return idx2[:, 0], dist2[:, 0]
